# baseline (speedup 1.0000x reference)
import functools

import jax
import jax.numpy as jnp
from jax.experimental import pallas as pl
from jax.experimental.pallas import tpu as pltpu


def _body(facs_ref, x_ref, noise_ref, o_ref, *, w, cutn):
    c_dim, _, rows_s, _ = x_ref.shape
    rows = c_dim * rows_s
    v = x_ref[...].astype(jnp.float32).reshape(rows, 2 * w)
    top = v[:, 0:w]
    bot = v[:, w:2 * w]
    rs = top + bot
    rm = jnp.maximum(top, bot)
    i = jax.lax.broadcasted_iota(jnp.int32, (w, w // 2), 0)
    jj = jax.lax.broadcasted_iota(jnp.int32, (w, w // 2), 1)
    e0 = (i == 2 * jj).astype(jnp.float32)
    e1 = (i == 2 * jj + 1).astype(jnp.float32)

    def dot(a, b):
        return jax.lax.dot_general(
            a, b, (((1,), (0,)), ((), ())),
            preferred_element_type=jnp.float32)

    def sel_dot(a, b):
        hi = a.astype(jnp.bfloat16).astype(jnp.float32)
        lo = a - hi
        return dot(hi, b) + dot(lo, b)

    cs_ = sel_dot(rs, e0 + e1)
    cm = jnp.maximum(sel_dot(rm, e0), sel_dot(rm, e1))
    pooled = ((cs_ * 0.25 + cm) * 0.5).reshape(c_dim, rows_s, w // 2)

    for b in range(cutn):
        fac = facs_ref[b]
        o_ref[b] = (pooled + fac * noise_ref[b].astype(jnp.float32)).astype(
            o_ref.dtype)


def kernel(x, facs, noise):
    N, C, H, W = x.shape
    cutn, _, cs, _ = noise.shape
    S = 4
    rows_s = cs // S
    x4 = x[0].reshape(C, S, rows_s, 2 * W)

    out = pl.pallas_call(
        functools.partial(_body, w=W, cutn=cutn),
        out_shape=jax.ShapeDtypeStruct((cutn, C, cs, cs), x.dtype),
        grid=(S,),
        in_specs=[
            pl.BlockSpec(memory_space=pltpu.MemorySpace.SMEM),
            pl.BlockSpec((C, 1, rows_s, 2 * W), lambda s: (0, s, 0, 0)),
            pl.BlockSpec((cutn, C, rows_s, cs), lambda s: (0, 0, s, 0)),
        ],
        out_specs=pl.BlockSpec((cutn, C, rows_s, cs), lambda s: (0, 0, s, 0)),
        compiler_params=pltpu.CompilerParams(
            dimension_semantics=("parallel",),
            vmem_limit_bytes=64 * 1024 * 1024,
        ),
    )(facs, x4, noise)

    return out
